# baseline (speedup 1.0000x reference)
import functools
import math

import jax
import jax.numpy as jnp
import numpy as np
from jax.experimental import pallas as pl
from jax.experimental.pallas import tpu as pltpu

N_NODES = 10000
HEADS = 4
HID = 256
ROW_BLOCK = 400


def _mm_body(x_ref, w_ref, b_ref, o_ref):
    o_ref[...] = (
        jnp.dot(x_ref[...], w_ref[...], preferred_element_type=jnp.float32)
        + b_ref[...]
    )


def _matmul_bias(x, w, b):
    n, k = x.shape
    p = w.shape[1]
    grid = (n // ROW_BLOCK,)
    return pl.pallas_call(
        _mm_body,
        grid=grid,
        in_specs=[
            pl.BlockSpec((ROW_BLOCK, k), lambda i: (i, 0)),
            pl.BlockSpec((k, p), lambda i: (0, 0)),
            pl.BlockSpec((1, p), lambda i: (0, 0)),
        ],
        out_specs=pl.BlockSpec((ROW_BLOCK, p), lambda i: (i, 0)),
        out_shape=jax.ShapeDtypeStruct((n, p), jnp.float32),
    )(x, w, b.reshape(1, p))


def _pad_w(w, b, mult=128):
    p = w.shape[1]
    pp = ((p + mult - 1) // mult) * mult
    if pp != p:
        w = jnp.pad(w, ((0, 0), (0, pp - p)))
        b = jnp.pad(b, ((0, pp - p),))
    return w, b


def _mm_lsm_body(x_ref, w_ref, b_ref, o_ref, *, p_real):
    z = (
        jnp.dot(x_ref[...], w_ref[...], preferred_element_type=jnp.float32)
        + b_ref[...]
    )
    mask = jax.lax.broadcasted_iota(jnp.int32, z.shape, 1) < p_real
    z = jnp.where(mask, z, -jnp.inf)
    m = jnp.max(z, axis=1, keepdims=True)
    lse = jnp.log(jnp.sum(jnp.exp(z - m), axis=1, keepdims=True)) + m
    o_ref[...] = z - lse


def _matmul_log_softmax(x, w, b, p_real):
    n, k = x.shape
    p = w.shape[1]
    grid = (n // ROW_BLOCK,)
    out = pl.pallas_call(
        functools.partial(_mm_lsm_body, p_real=p_real),
        grid=grid,
        in_specs=[
            pl.BlockSpec((ROW_BLOCK, k), lambda i: (i, 0)),
            pl.BlockSpec((k, p), lambda i: (0, 0)),
            pl.BlockSpec((1, p), lambda i: (0, 0)),
        ],
        out_specs=pl.BlockSpec((ROW_BLOCK, p), lambda i: (i, 0)),
        out_shape=jax.ShapeDtypeStruct((n, p), jnp.float32),
    )(x, w, b.reshape(1, p))
    return out[:, :p_real]


def _gln_body(x_ref, g_ref, b_ref, o_ref):
    x = x_ref[...]
    mu = jnp.mean(x)
    var = jnp.mean((x - mu) ** 2)
    o_ref[...] = (x - mu) / jnp.sqrt(var + 1e-5) * g_ref[...] + b_ref[...]


def _gln(p, x):
    n, d = x.shape
    return pl.pallas_call(
        _gln_body,
        in_specs=[
            pl.BlockSpec((n, d), lambda: (0, 0)),
            pl.BlockSpec((1, d), lambda: (0, 0)),
            pl.BlockSpec((1, d), lambda: (0, 0)),
        ],
        out_specs=pl.BlockSpec((n, d), lambda: (0, 0)),
        out_shape=jax.ShapeDtypeStruct((n, d), jnp.float32),
    )(x, p["g"].reshape(1, d), p["b"].reshape(1, d))


def _seg_mean(x_src_gathered_sum, cnt):
    return x_src_gathered_sum / jnp.maximum(cnt, 1.0)[:, None]


def _sage(p, x_src, x_dst, ei, num_dst):
    src, dst = ei[0], ei[1]
    s = jax.ops.segment_sum(x_src[src], dst, num_segments=num_dst)
    cnt = jax.ops.segment_sum(
        jnp.ones((ei.shape[1],), jnp.float32), dst, num_segments=num_dst
    )
    agg = _seg_mean(s, cnt)
    wcat = jnp.concatenate([p["l"]["W"], p["Wr"]], axis=0)
    return _matmul_bias(jnp.concatenate([agg, x_dst], axis=1), wcat, p["l"]["b"])


def _tconv(p, x_src, x_dst, ei, heads, dout, num_dst):
    src, dst = ei[0], ei[1]
    q = _matmul_bias(x_dst, p["q"]["W"], p["q"]["b"]).reshape(-1, heads, dout)
    k = _matmul_bias(x_src, p["k"]["W"], p["k"]["b"]).reshape(-1, heads, dout)
    v = _matmul_bias(x_src, p["v"]["W"], p["v"]["b"]).reshape(-1, heads, dout)
    skip = _matmul_bias(x_dst, p["skip"]["W"], p["skip"]["b"])
    logits = (q[dst] * k[src]).sum(-1) / np.sqrt(dout)
    m = jax.ops.segment_max(logits, dst, num_segments=num_dst)
    ex = jnp.exp(logits - m[dst])
    den = jax.ops.segment_sum(ex, dst, num_segments=num_dst)
    alpha = ex / (den[dst] + 1e-16)
    out = jax.ops.segment_sum(
        v[src] * alpha[:, :, None], dst, num_segments=num_dst
    )
    return out.reshape(num_dst, heads * dout) + skip


def _attention(p, x, ei, n):
    h = _tconv(p["att"], x, x, ei, HEADS, HID, n)
    return _gln(p["norm"], _matmul_bias(h, p["cat"]["W"], p["cat"]["b"]))


def _ffw(p, x, ei, n):
    return _gln(p["norm"], _sage(p["sage"], x, x, ei, n))


def kernel(ast_x, llc_x, params, ast_edge_index, llc_edge_index):
    n = N_NODES
    x = _sage(params["enc"]["embed"], llc_x, llc_x, llc_edge_index, n)
    for u in params["enc"]["units"]:
        x = _attention(u["att"], x, llc_edge_index, n)
        x = _ffw(u["ffw"], x, llc_edge_index, n)
    enc_out = x

    y = _sage(params["dec"]["embed"], ast_x, ast_x, ast_edge_index, n)
    for u in params["dec"]["units"]:
        y = _attention(u["ast_att"], y, ast_edge_index, n)
        y = _tconv(u["cross"], y, enc_out, ast_edge_index, HEADS, HID, n)
        y = _gln(u["norm"], _matmul_bias(y, u["cat"]["W"], u["cat"]["b"]))
        y = _ffw(u["ffw"], y, ast_edge_index, n)

    nn_w, nn_b = _pad_w(params["new_node"]["W"], params["new_node"]["b"])
    new_node = _matmul_log_softmax(y, nn_w, nn_b, params["new_node"]["W"].shape[1])

    ns = params["node_sel"]
    src, dst = ast_edge_index[0], ast_edge_index[1]
    s = jax.ops.segment_sum(y[src], dst, num_segments=n)
    cnt = jax.ops.segment_sum(
        jnp.ones((ast_edge_index.shape[1],), jnp.float32), dst, num_segments=n
    )
    agg = _seg_mean(s, cnt)
    wcat = jnp.concatenate([ns["l"]["W"], ns["Wr"]], axis=0)
    wcat, bp = _pad_w(wcat, ns["l"]["b"])
    node_sel = _matmul_bias(jnp.concatenate([agg, y], axis=1), wcat, bp)[:, :1]
    return new_node, node_sel


# speedup vs baseline: 1.0263x; 1.0263x over previous
import functools
import math

import jax
import jax.numpy as jnp
import numpy as np
from jax import lax
from jax.experimental import pallas as pl
from jax.experimental.pallas import tpu as pltpu
from jax.experimental.pallas import tpu_sc as plsc

N_NODES = 10000
HEADS = 4
HID = 256
ROW_BLOCK = 400

E_TOTAL = 160000
SAGE_B = 80
CNT_B = 40
_SC_MESH = dict(core_axis_name="c", subcore_axis_name="s")


def _zero_fill(ref, rows, cols):
    z = jnp.zeros((16,), jnp.float32)
    for r in range(rows):
        for g in range(cols // 16):
            ref[r, g * 16:(g + 1) * 16] = z


def _zero_spmem(z128, dst_spmem, sid, n_rows):
    n_blk = n_rows // 16
    def zbody(i, _):
        b = sid + i * 16
        @pl.when(b < n_blk)
        def _():
            pltpu.sync_copy(z128, dst_spmem.at[pl.ds(b * 16, 16)])
        return 0
    lax.fori_loop(0, (n_blk + 15) // 16, zbody, 0)


def _spmem_writeback(acc, out_hbm, sid):
    rows_per = 624
    @pl.when(sid < 15)
    def _():
        pltpu.sync_copy(acc.at[pl.ds(sid * rows_per, rows_per)],
                        out_hbm.at[pl.ds(sid * rows_per, rows_per)])
    @pl.when(sid == 15)
    def _():
        tail = N_NODES - 15 * rows_per
        pltpu.sync_copy(acc.at[pl.ds(15 * rows_per, tail)],
                        out_hbm.at[pl.ds(15 * rows_per, tail)])


def _sage_agg_body(x2_hbm, src_hbm, dst_hbm, out_hbm,
                   srcb, dstb, rows, z128, acc, sem):
    cid = lax.axis_index("c")
    sid = lax.axis_index("s")

    _zero_fill(z128, 16, 128)
    _zero_spmem(z128, acc, sid, N_NODES)
    plsc.subcore_barrier()

    per_tile = E_TOTAL // 16
    n_batch = per_tile // SAGE_B

    def body(i, _):
        base = sid * per_tile + i * SAGE_B
        pltpu.sync_copy(src_hbm.at[pl.ds(base, SAGE_B)], srcb)
        pltpu.sync_copy(dst_hbm.at[pl.ds(base, SAGE_B)], dstb)
        pltpu.async_copy(x2_hbm.at[cid].at[srcb], rows, sem).wait()
        pltpu.async_copy(rows, acc.at[dstb], sem, add=True).wait()
        return 0
    lax.fori_loop(0, n_batch, body, 0)
    plsc.subcore_barrier()
    _spmem_writeback(acc, out_hbm.at[cid], sid)


@functools.partial(
    pl.kernel,
    mesh=plsc.VectorSubcoreMesh(**_SC_MESH),
    out_type=jax.ShapeDtypeStruct((2, N_NODES, 128), jnp.float32),
    scratch_types=[
        pltpu.VMEM((SAGE_B,), jnp.int32),
        pltpu.VMEM((SAGE_B,), jnp.int32),
        pltpu.VMEM((SAGE_B, 128), jnp.float32),
        pltpu.VMEM((16, 128), jnp.float32),
        pltpu.VMEM_SHARED((N_NODES, 128), jnp.float32),
        pltpu.SemaphoreType.DMA,
    ],
)
def _sage_agg(x2_hbm, src_hbm, dst_hbm, out_hbm, *scratch):
    _sage_agg_body(x2_hbm, src_hbm, dst_hbm, out_hbm, *scratch)


def _cnt_body(dst_hbm, out_hbm, dstb, ones_b, z128, acc, sem):
    cid = lax.axis_index("c")
    sid = lax.axis_index("s")

    _zero_fill(z128, 16, 128)
    one_row = jnp.where(
        lax.broadcasted_iota(jnp.int32, (16,), 0) == 0, 1.0, 0.0
    ).astype(jnp.float32)
    zero16 = jnp.zeros((16,), jnp.float32)
    for r in range(CNT_B):
        for g in range(8):
            ones_b[r, g * 16:(g + 1) * 16] = one_row if g == 0 else zero16
    _zero_spmem(z128, acc, sid, N_NODES)
    plsc.subcore_barrier()

    per_sc = E_TOTAL // 2
    per_tile = per_sc // 16
    n_batch = per_tile // CNT_B

    def body(i, _):
        base = cid * per_sc + sid * per_tile + i * CNT_B
        pltpu.sync_copy(dst_hbm.at[pl.ds(base, CNT_B)], dstb)
        pltpu.async_copy(ones_b, acc.at[dstb], sem, add=True).wait()
        return 0
    lax.fori_loop(0, n_batch, body, 0)
    plsc.subcore_barrier()
    _spmem_writeback(acc, out_hbm.at[cid], sid)


@functools.partial(
    pl.kernel,
    mesh=plsc.VectorSubcoreMesh(**_SC_MESH),
    out_type=jax.ShapeDtypeStruct((2, N_NODES, 128), jnp.float32),
    scratch_types=[
        pltpu.VMEM((CNT_B,), jnp.int32),
        pltpu.VMEM((CNT_B, 128), jnp.float32),
        pltpu.VMEM((16, 128), jnp.float32),
        pltpu.VMEM_SHARED((N_NODES, 128), jnp.float32),
        pltpu.SemaphoreType.DMA,
    ],
)
def _cnt_kernel(dst_hbm, out_hbm, *scratch):
    _cnt_body(dst_hbm, out_hbm, *scratch)


def _mm_body(x_ref, w_ref, b_ref, o_ref):
    o_ref[...] = (
        jnp.dot(x_ref[...], w_ref[...], preferred_element_type=jnp.float32)
        + b_ref[...]
    )


def _matmul_bias(x, w, b):
    n, k = x.shape
    p = w.shape[1]
    grid = (n // ROW_BLOCK,)
    return pl.pallas_call(
        _mm_body,
        grid=grid,
        in_specs=[
            pl.BlockSpec((ROW_BLOCK, k), lambda i: (i, 0)),
            pl.BlockSpec((k, p), lambda i: (0, 0)),
            pl.BlockSpec((1, p), lambda i: (0, 0)),
        ],
        out_specs=pl.BlockSpec((ROW_BLOCK, p), lambda i: (i, 0)),
        out_shape=jax.ShapeDtypeStruct((n, p), jnp.float32),
    )(x, w, b.reshape(1, p))


def _pad_w(w, b, mult=128):
    p = w.shape[1]
    pp = ((p + mult - 1) // mult) * mult
    if pp != p:
        w = jnp.pad(w, ((0, 0), (0, pp - p)))
        b = jnp.pad(b, ((0, pp - p),))
    return w, b


def _mm_lsm_body(x_ref, w_ref, b_ref, o_ref, *, p_real):
    z = (
        jnp.dot(x_ref[...], w_ref[...], preferred_element_type=jnp.float32)
        + b_ref[...]
    )
    mask = jax.lax.broadcasted_iota(jnp.int32, z.shape, 1) < p_real
    z = jnp.where(mask, z, -jnp.inf)
    m = jnp.max(z, axis=1, keepdims=True)
    lse = jnp.log(jnp.sum(jnp.exp(z - m), axis=1, keepdims=True)) + m
    o_ref[...] = z - lse


def _matmul_log_softmax(x, w, b, p_real):
    n, k = x.shape
    p = w.shape[1]
    grid = (n // ROW_BLOCK,)
    out = pl.pallas_call(
        functools.partial(_mm_lsm_body, p_real=p_real),
        grid=grid,
        in_specs=[
            pl.BlockSpec((ROW_BLOCK, k), lambda i: (i, 0)),
            pl.BlockSpec((k, p), lambda i: (0, 0)),
            pl.BlockSpec((1, p), lambda i: (0, 0)),
        ],
        out_specs=pl.BlockSpec((ROW_BLOCK, p), lambda i: (i, 0)),
        out_shape=jax.ShapeDtypeStruct((n, p), jnp.float32),
    )(x, w, b.reshape(1, p))
    return out[:, :p_real]


def _gln_body(x_ref, g_ref, b_ref, o_ref):
    x = x_ref[...]
    mu = jnp.mean(x)
    var = jnp.mean((x - mu) ** 2)
    o_ref[...] = (x - mu) / jnp.sqrt(var + 1e-5) * g_ref[...] + b_ref[...]


def _gln(p, x):
    n, d = x.shape
    return pl.pallas_call(
        _gln_body,
        in_specs=[
            pl.BlockSpec((n, d), lambda: (0, 0)),
            pl.BlockSpec((1, d), lambda: (0, 0)),
            pl.BlockSpec((1, d), lambda: (0, 0)),
        ],
        out_specs=pl.BlockSpec((n, d), lambda: (0, 0)),
        out_shape=jax.ShapeDtypeStruct((n, d), jnp.float32),
    )(x, p["g"].reshape(1, d), p["b"].reshape(1, d))


def _split2_body(x_ref, o_ref):
    o_ref[0, :, :] = x_ref[:, 0:128]
    o_ref[1, :, :] = x_ref[:, 128:256]


def _split2(x):
    n = x.shape[0]
    return pl.pallas_call(
        _split2_body,
        grid=(n // ROW_BLOCK,),
        in_specs=[pl.BlockSpec((ROW_BLOCK, 256), lambda i: (i, 0))],
        out_specs=pl.BlockSpec((2, ROW_BLOCK, 128), lambda i: (0, i, 0)),
        out_shape=jax.ShapeDtypeStruct((2, n, 128), jnp.float32),
    )(x)


def _sage_lin_body(s_ref, c_ref, x_ref, wl_ref, wr_ref, b_ref, o_ref):
    cnt = c_ref[0, :, 0:1] + c_ref[1, :, 0:1]
    inv = 1.0 / jnp.maximum(cnt, 1.0)
    agg = jnp.concatenate([s_ref[0], s_ref[1]], axis=1) * inv
    o_ref[...] = (
        jnp.dot(agg, wl_ref[...], preferred_element_type=jnp.float32)
        + jnp.dot(x_ref[...], wr_ref[...], preferred_element_type=jnp.float32)
        + b_ref[...]
    )


def _sage_linear(s2, cnt2, x, wl, wr, b):
    n, k = x.shape
    p = wl.shape[1]
    grid = (n // ROW_BLOCK,)
    return pl.pallas_call(
        _sage_lin_body,
        grid=grid,
        in_specs=[
            pl.BlockSpec((2, ROW_BLOCK, 128), lambda i: (0, i, 0)),
            pl.BlockSpec((2, ROW_BLOCK, 128), lambda i: (0, i, 0)),
            pl.BlockSpec((ROW_BLOCK, k), lambda i: (i, 0)),
            pl.BlockSpec((k, p), lambda i: (0, 0)),
            pl.BlockSpec((k, p), lambda i: (0, 0)),
            pl.BlockSpec((1, p), lambda i: (0, 0)),
        ],
        out_specs=pl.BlockSpec((ROW_BLOCK, p), lambda i: (i, 0)),
        out_shape=jax.ShapeDtypeStruct((n, p), jnp.float32),
    )(s2, cnt2, x, wl, wr, b.reshape(1, p))


def _sage(p, x_src, x_dst, ei, num_dst, cnt2, x2=None):
    if x2 is None:
        x2 = _split2(x_src)
    s2 = _sage_agg(x2, ei[0], ei[1])
    return _sage_linear(s2, cnt2, x_dst, p["l"]["W"], p["Wr"], p["l"]["b"])


def _tconv(p, x_src, x_dst, ei, heads, dout, num_dst):
    src, dst = ei[0], ei[1]
    q = _matmul_bias(x_dst, p["q"]["W"], p["q"]["b"]).reshape(-1, heads, dout)
    k = _matmul_bias(x_src, p["k"]["W"], p["k"]["b"]).reshape(-1, heads, dout)
    v = _matmul_bias(x_src, p["v"]["W"], p["v"]["b"]).reshape(-1, heads, dout)
    skip = _matmul_bias(x_dst, p["skip"]["W"], p["skip"]["b"])
    logits = (q[dst] * k[src]).sum(-1) / np.sqrt(dout)
    m = jax.ops.segment_max(logits, dst, num_segments=num_dst)
    ex = jnp.exp(logits - m[dst])
    den = jax.ops.segment_sum(ex, dst, num_segments=num_dst)
    alpha = ex / (den[dst] + 1e-16)
    out = jax.ops.segment_sum(
        v[src] * alpha[:, :, None], dst, num_segments=num_dst
    )
    return out.reshape(num_dst, heads * dout) + skip


def _attention(p, x, ei, n):
    h = _tconv(p["att"], x, x, ei, HEADS, HID, n)
    return _gln(p["norm"], _matmul_bias(h, p["cat"]["W"], p["cat"]["b"]))


def _ffw(p, x, ei, n, cnt2):
    return _gln(p["norm"], _sage(p["sage"], x, x, ei, n, cnt2))


def kernel(ast_x, llc_x, params, ast_edge_index, llc_edge_index):
    n = N_NODES
    cnt_llc = _cnt_kernel(llc_edge_index[1])
    cnt_ast = _cnt_kernel(ast_edge_index[1])

    x = _sage(params["enc"]["embed"], llc_x, llc_x, llc_edge_index, n, cnt_llc)
    for u in params["enc"]["units"]:
        x = _attention(u["att"], x, llc_edge_index, n)
        x = _ffw(u["ffw"], x, llc_edge_index, n, cnt_llc)
    enc_out = x

    y = _sage(params["dec"]["embed"], ast_x, ast_x, ast_edge_index, n, cnt_ast)
    for u in params["dec"]["units"]:
        y = _attention(u["ast_att"], y, ast_edge_index, n)
        y = _tconv(u["cross"], y, enc_out, ast_edge_index, HEADS, HID, n)
        y = _gln(u["norm"], _matmul_bias(y, u["cat"]["W"], u["cat"]["b"]))
        y = _ffw(u["ffw"], y, ast_edge_index, n, cnt_ast)

    nn_w, nn_b = _pad_w(params["new_node"]["W"], params["new_node"]["b"])
    new_node = _matmul_log_softmax(y, nn_w, nn_b, params["new_node"]["W"].shape[1])

    ns = params["node_sel"]
    s2 = _sage_agg(_split2(y), ast_edge_index[0], ast_edge_index[1])
    wl, bp = _pad_w(ns["l"]["W"], ns["l"]["b"])
    wr, _ = _pad_w(ns["Wr"], ns["l"]["b"])
    node_sel = _sage_linear(s2, cnt_ast, y, wl, wr, bp)[:, :1]
    return new_node, node_sel


# speedup vs baseline: 2.5682x; 2.5023x over previous
import functools
import math

import jax
import jax.numpy as jnp
import numpy as np
from jax import lax
from jax.experimental import pallas as pl
from jax.experimental.pallas import tpu as pltpu
from jax.experimental.pallas import tpu_sc as plsc

N_NODES = 10000
HEADS = 4
HID = 256
ROW_BLOCK = 400

E_TOTAL = 160000
SAGE_B = 80
CNT_B = 40
_SC_MESH = dict(core_axis_name="c", subcore_axis_name="s")
_SC_NLP = pltpu.CompilerParams(needs_layout_passes=False)


def _zero_fill(ref, rows, cols):
    z = jnp.zeros((16,), jnp.float32)
    for r in range(rows):
        for g in range(cols // 16):
            ref[r, g * 16:(g + 1) * 16] = z


def _zero_spmem(z128, dst_spmem, sid, n_rows):
    n_blk = n_rows // 16
    def zbody(i, _):
        b = sid + i * 16
        @pl.when(b < n_blk)
        def _():
            pltpu.sync_copy(z128, dst_spmem.at[pl.ds(b * 16, 16)])
        return 0
    lax.fori_loop(0, (n_blk + 15) // 16, zbody, 0)


def _spmem_writeback(acc, out_hbm, sid):
    rows_per = 624
    @pl.when(sid < 15)
    def _():
        pltpu.sync_copy(acc.at[pl.ds(sid * rows_per, rows_per)],
                        out_hbm.at[pl.ds(sid * rows_per, rows_per)])
    @pl.when(sid == 15)
    def _():
        tail = N_NODES - 15 * rows_per
        pltpu.sync_copy(acc.at[pl.ds(15 * rows_per, tail)],
                        out_hbm.at[pl.ds(15 * rows_per, tail)])


def _sage_agg_body(x2_hbm, src_hbm, dst_hbm, out_hbm,
                   srcb, dstb, rows, z128, acc, sem):
    cid = lax.axis_index("c")
    sid = lax.axis_index("s")

    _zero_fill(z128, 16, 128)
    _zero_spmem(z128, acc, sid, N_NODES)
    plsc.subcore_barrier()

    per_tile = E_TOTAL // 16
    n_batch = per_tile // SAGE_B

    def body(i, _):
        base = sid * per_tile + i * SAGE_B
        pltpu.sync_copy(src_hbm.at[pl.ds(base, SAGE_B)], srcb)
        pltpu.sync_copy(dst_hbm.at[pl.ds(base, SAGE_B)], dstb)
        pltpu.async_copy(x2_hbm.at[cid].at[srcb], rows, sem).wait()
        pltpu.async_copy(rows, acc.at[dstb], sem, add=True).wait()
        return 0
    lax.fori_loop(0, n_batch, body, 0)
    plsc.subcore_barrier()
    _spmem_writeback(acc, out_hbm.at[cid], sid)


@functools.partial(
    pl.kernel,
    mesh=plsc.VectorSubcoreMesh(**_SC_MESH),
    out_type=jax.ShapeDtypeStruct((2, N_NODES, 128), jnp.float32),
    scratch_types=[
        pltpu.VMEM((SAGE_B,), jnp.int32),
        pltpu.VMEM((SAGE_B,), jnp.int32),
        pltpu.VMEM((SAGE_B, 128), jnp.float32),
        pltpu.VMEM((16, 128), jnp.float32),
        pltpu.VMEM_SHARED((N_NODES, 128), jnp.float32),
        pltpu.SemaphoreType.DMA,
    ],
    compiler_params=_SC_NLP,
)
def _sage_agg(x2_hbm, src_hbm, dst_hbm, out_hbm, *scratch):
    _sage_agg_body(x2_hbm, src_hbm, dst_hbm, out_hbm, *scratch)


def _cnt_body(dst_hbm, out_hbm, dstb, ones_b, z128, acc, sem):
    cid = lax.axis_index("c")
    sid = lax.axis_index("s")

    _zero_fill(z128, 16, 128)
    one_row = jnp.where(
        lax.broadcasted_iota(jnp.int32, (16,), 0) == 0, 1.0, 0.0
    ).astype(jnp.float32)
    zero16 = jnp.zeros((16,), jnp.float32)
    for r in range(CNT_B):
        for g in range(8):
            ones_b[r, g * 16:(g + 1) * 16] = one_row if g == 0 else zero16
    _zero_spmem(z128, acc, sid, N_NODES)
    plsc.subcore_barrier()

    per_sc = E_TOTAL // 2
    per_tile = per_sc // 16
    n_batch = per_tile // CNT_B

    def body(i, _):
        base = cid * per_sc + sid * per_tile + i * CNT_B
        pltpu.sync_copy(dst_hbm.at[pl.ds(base, CNT_B)], dstb)
        pltpu.async_copy(ones_b, acc.at[dstb], sem, add=True).wait()
        return 0
    lax.fori_loop(0, n_batch, body, 0)
    plsc.subcore_barrier()
    _spmem_writeback(acc, out_hbm.at[cid], sid)


@functools.partial(
    pl.kernel,
    mesh=plsc.VectorSubcoreMesh(**_SC_MESH),
    out_type=jax.ShapeDtypeStruct((2, N_NODES, 128), jnp.float32),
    scratch_types=[
        pltpu.VMEM((CNT_B,), jnp.int32),
        pltpu.VMEM((CNT_B, 128), jnp.float32),
        pltpu.VMEM((16, 128), jnp.float32),
        pltpu.VMEM_SHARED((N_NODES, 128), jnp.float32),
        pltpu.SemaphoreType.DMA,
    ],
    compiler_params=_SC_NLP,
)
def _cnt_kernel(dst_hbm, out_hbm, *scratch):
    _cnt_body(dst_hbm, out_hbm, *scratch)


EA_NBATCH = E_TOTAL // 16
EA_FULL = EA_NBATCH // 32
TBL = 40960
NEG = -3.0e38


def _attn_logits_body(q8, k8, src_hbm, dst_hbm, lg_hbm, tb_hbm,
                      srcb, dstb, qb, kb, lbuf, lrow, tbl, sem):
    cid = lax.axis_index("c")
    sid = lax.axis_index("s")
    wid = 16 * cid + sid
    iota = lax.broadcasted_iota(jnp.int32, (16,), 0)

    neg = jnp.full((16,), NEG, jnp.float32)
    def tinit(i, _):
        plsc.store_scatter(tbl, [i * 16 + iota], neg)
        return 0
    lax.fori_loop(0, TBL // 16, tinit, 0)

    m4 = iota & 3

    def do_batch(b):
        base = b * 16
        pltpu.sync_copy(src_hbm.at[pl.ds(base, 16)], srcb)
        pltpu.sync_copy(dst_hbm.at[pl.ds(base, 16)], dstb)
        cps = []
        for blk in range(4):
            cps.append(pltpu.async_copy(q8.at[blk].at[dstb], qb.at[blk], sem))
            cps.append(pltpu.async_copy(k8.at[blk].at[srcb], kb.at[blk], sem))
        for cp in cps:
            cp.wait()
        dstv = dstb[pl.ds(0, 16)]
        iota128 = iota * 128
        for h in range(4):
            acc = jnp.zeros((16,), jnp.float32)
            for half in range(2):
                blk = 2 * h + half
                for d in range(128):
                    qv = plsc.load_gather(qb, [jnp.full((16,), blk, jnp.int32),
                                               iota, jnp.full((16,), d, jnp.int32)])
                    kv = plsc.load_gather(kb, [jnp.full((16,), blk, jnp.int32),
                                               iota, jnp.full((16,), d, jnp.int32)])
                    acc = acc + qv * kv
            plsc.store_scatter(lbuf, [h * 16 + iota], acc)
        for h in range(4):
            lcol = plsc.load_gather(lbuf, [h * 16 + iota])
            plsc.store_scatter(lrow, [iota, jnp.full((16,), h, jnp.int32)], lcol)
        lmask = iota < 4
        for j in range(16):
            dj = dstv[j]
            lvj = plsc.load_gather(lbuf, [iota * 16 + j], mask=lmask)
            tix = dj * 4 + m4
            cur = plsc.load_gather(tbl, [tix], mask=lmask)
            plsc.store_scatter(tbl, [tix], jnp.maximum(cur, lvj), mask=lmask)
        pltpu.sync_copy(lrow, lg_hbm.at[pl.ds(base, 16)])

    def body(i, _):
        do_batch(wid + 32 * i)
        return 0
    lax.fori_loop(0, EA_FULL, body, 0)
    @pl.when(wid < EA_NBATCH - 32 * EA_FULL)
    def _():
        do_batch(wid + 32 * EA_FULL)

    pltpu.sync_copy(tbl, tb_hbm.at[wid])


@functools.partial(
    pl.kernel,
    mesh=plsc.VectorSubcoreMesh(**_SC_MESH),
    out_type=[
        jax.ShapeDtypeStruct((E_TOTAL + 16, 8), jnp.float32),
        jax.ShapeDtypeStruct((32, TBL), jnp.float32),
    ],
    scratch_types=[
        pltpu.VMEM((16,), jnp.int32),
        pltpu.VMEM((16,), jnp.int32),
        pltpu.VMEM((4, 16, 128), jnp.float32),
        pltpu.VMEM((4, 16, 128), jnp.float32),
        pltpu.VMEM((64,), jnp.float32),
        pltpu.VMEM((16, 8), jnp.float32),
        pltpu.VMEM((TBL,), jnp.float32),
        pltpu.SemaphoreType.DMA,
    ],
    compiler_params=_SC_NLP,
)
def _attn_logits(q8, k8, src_hbm, dst_hbm, lg_hbm, tb_hbm, *scratch):
    _attn_logits_body(q8, k8, src_hbm, dst_hbm, lg_hbm, tb_hbm, *scratch)


def _attn_den_body(lg_hbm, m_hbm, src_hbm, dst_hbm, ex_hbm, den_hbm,
                   dstb, lrow, exrow, ex128, mtbl, z128, acc, sem):
    cid = lax.axis_index("c")
    sid = lax.axis_index("s")
    wid = 16 * cid + sid
    iota = lax.broadcasted_iota(jnp.int32, (16,), 0)

    _zero_fill(z128, 16, 128)
    _zero_spmem(z128, acc, sid, N_NODES + 16)
    zero16 = jnp.zeros((16,), jnp.float32)
    for r in range(16):
        for g in range(8):
            ex128[r, g * 16:(g + 1) * 16] = zero16
    pltpu.sync_copy(m_hbm.at[0], mtbl)
    plsc.subcore_barrier()

    def do_batch(b):
        base = b * 16
        pltpu.sync_copy(dst_hbm.at[pl.ds(base, 16)], dstb)
        pltpu.sync_copy(lg_hbm.at[pl.ds(base, 16)], lrow)
        dstv = dstb[pl.ds(0, 16)]
        for h in range(4):
            hc = jnp.full((16,), h, jnp.int32)
            lcol = plsc.load_gather(lrow, [iota, hc])
            mcol = plsc.load_gather(mtbl, [dstv * 4 + h])
            exc = jnp.exp(lcol - mcol)
            plsc.store_scatter(exrow, [iota, hc], exc)
            plsc.store_scatter(ex128, [iota, hc], exc)
        pltpu.sync_copy(exrow, ex_hbm.at[pl.ds(base, 16)])
        pltpu.async_copy(ex128, acc.at[dstb], sem, add=True).wait()

    def body(i, _):
        do_batch(wid + 32 * i)
        return 0
    lax.fori_loop(0, EA_FULL, body, 0)
    @pl.when(wid < EA_NBATCH - 32 * EA_FULL)
    def _():
        do_batch(wid + 32 * EA_FULL)

    plsc.subcore_barrier()
    _spmem_writeback(acc, den_hbm.at[cid], sid)


@functools.partial(
    pl.kernel,
    mesh=plsc.VectorSubcoreMesh(**_SC_MESH),
    out_type=[
        jax.ShapeDtypeStruct((E_TOTAL + 16, 8), jnp.float32),
        jax.ShapeDtypeStruct((2, N_NODES, 128), jnp.float32),
    ],
    scratch_types=[
        pltpu.VMEM((16,), jnp.int32),
        pltpu.VMEM((16, 8), jnp.float32),
        pltpu.VMEM((16, 8), jnp.float32),
        pltpu.VMEM((16, 128), jnp.float32),
        pltpu.VMEM((TBL,), jnp.float32),
        pltpu.VMEM((16, 128), jnp.float32),
        pltpu.VMEM_SHARED((N_NODES + 16, 128), jnp.float32),
        pltpu.SemaphoreType.DMA,
    ],
    compiler_params=_SC_NLP,
)
def _attn_den(lg_hbm, m_hbm, src_hbm, dst_hbm, ex_hbm, den_hbm, *scratch):
    _attn_den_body(lg_hbm, m_hbm, src_hbm, dst_hbm, ex_hbm, den_hbm, *scratch)


EC_B = 80


def _attn_vagg_body(v8, ex_hbm, den_hbm, src_hbm, dst_hbm, out8,
                    srcb, dstb, rows, exb, denb, z128, acc, sem):
    cid = lax.axis_index("c")
    sid = lax.axis_index("s")
    iota = lax.broadcasted_iota(jnp.int32, (16,), 0)

    _zero_fill(z128, 16, 128)
    per_tile = E_TOTAL // 16
    n_batch = per_tile // EC_B

    for p in range(4):
        blk = 4 * cid + p
        h = blk // 2
        _zero_spmem(z128, acc, sid, N_NODES)
        plsc.subcore_barrier()

        def body(i, _):
            base = sid * per_tile + i * EC_B
            pltpu.sync_copy(src_hbm.at[pl.ds(base, EC_B)], srcb)
            pltpu.sync_copy(dst_hbm.at[pl.ds(base, EC_B)], dstb)
            cp1 = pltpu.async_copy(v8.at[blk].at[srcb], rows, sem)
            pltpu.sync_copy(ex_hbm.at[pl.ds(base, EC_B)], exb)
            cp2 = pltpu.async_copy(den_hbm.at[dstb], denb, sem)
            cp1.wait()
            cp2.wait()
            for g in range(5):
                lane = g * 16 + iota
                exh = plsc.load_gather(exb, [lane, jnp.full((16,), h, jnp.int32)])
                dnh = plsc.load_gather(denb, [lane, jnp.full((16,), h, jnp.int32)])
                alpha = exh / (dnh + 1e-16)
                for j in range(16):
                    r = g * 16 + j
                    s = alpha[j]
                    for cg in range(8):
                        rows[r, cg * 16:(cg + 1) * 16] = (
                            rows[r, cg * 16:(cg + 1) * 16] * s
                        )
            pltpu.async_copy(rows, acc.at[dstb], sem, add=True).wait()
            return 0
        lax.fori_loop(0, n_batch, body, 0)
        plsc.subcore_barrier()
        _spmem_writeback(acc, out8.at[blk], sid)
        plsc.subcore_barrier()


@functools.partial(
    pl.kernel,
    mesh=plsc.VectorSubcoreMesh(**_SC_MESH),
    out_type=jax.ShapeDtypeStruct((8, N_NODES, 128), jnp.float32),
    scratch_types=[
        pltpu.VMEM((EC_B,), jnp.int32),
        pltpu.VMEM((EC_B,), jnp.int32),
        pltpu.VMEM((EC_B, 128), jnp.float32),
        pltpu.VMEM((EC_B, 8), jnp.float32),
        pltpu.VMEM((EC_B, 128), jnp.float32),
        pltpu.VMEM((16, 128), jnp.float32),
        pltpu.VMEM_SHARED((N_NODES, 128), jnp.float32),
        pltpu.SemaphoreType.DMA,
    ],
    compiler_params=_SC_NLP,
)
def _attn_vagg(v8, ex_hbm, den_hbm, src_hbm, dst_hbm, out8, *scratch):
    _attn_vagg_body(v8, ex_hbm, den_hbm, src_hbm, dst_hbm, out8, *scratch)


def _mm_body(x_ref, w_ref, b_ref, o_ref):
    o_ref[...] = (
        jnp.dot(x_ref[...], w_ref[...], preferred_element_type=jnp.float32)
        + b_ref[...]
    )


def _matmul_bias(x, w, b):
    n, k = x.shape
    p = w.shape[1]
    grid = (n // ROW_BLOCK,)
    return pl.pallas_call(
        _mm_body,
        grid=grid,
        in_specs=[
            pl.BlockSpec((ROW_BLOCK, k), lambda i: (i, 0)),
            pl.BlockSpec((k, p), lambda i: (0, 0)),
            pl.BlockSpec((1, p), lambda i: (0, 0)),
        ],
        out_specs=pl.BlockSpec((ROW_BLOCK, p), lambda i: (i, 0)),
        out_shape=jax.ShapeDtypeStruct((n, p), jnp.float32),
    )(x, w, b.reshape(1, p))


def _pad_w(w, b, mult=128):
    p = w.shape[1]
    pp = ((p + mult - 1) // mult) * mult
    if pp != p:
        w = jnp.pad(w, ((0, 0), (0, pp - p)))
        b = jnp.pad(b, ((0, pp - p),))
    return w, b


def _mm_lsm_body(x_ref, w_ref, b_ref, o_ref, *, p_real):
    z = (
        jnp.dot(x_ref[...], w_ref[...], preferred_element_type=jnp.float32)
        + b_ref[...]
    )
    mask = jax.lax.broadcasted_iota(jnp.int32, z.shape, 1) < p_real
    z = jnp.where(mask, z, -jnp.inf)
    m = jnp.max(z, axis=1, keepdims=True)
    lse = jnp.log(jnp.sum(jnp.exp(z - m), axis=1, keepdims=True)) + m
    o_ref[...] = z - lse


def _matmul_log_softmax(x, w, b, p_real):
    n, k = x.shape
    p = w.shape[1]
    grid = (n // ROW_BLOCK,)
    out = pl.pallas_call(
        functools.partial(_mm_lsm_body, p_real=p_real),
        grid=grid,
        in_specs=[
            pl.BlockSpec((ROW_BLOCK, k), lambda i: (i, 0)),
            pl.BlockSpec((k, p), lambda i: (0, 0)),
            pl.BlockSpec((1, p), lambda i: (0, 0)),
        ],
        out_specs=pl.BlockSpec((ROW_BLOCK, p), lambda i: (i, 0)),
        out_shape=jax.ShapeDtypeStruct((n, p), jnp.float32),
    )(x, w, b.reshape(1, p))
    return out[:, :p_real]


def _gln_body(x_ref, g_ref, b_ref, o_ref):
    x = x_ref[...]
    mu = jnp.mean(x)
    var = jnp.mean((x - mu) ** 2)
    o_ref[...] = (x - mu) / jnp.sqrt(var + 1e-5) * g_ref[...] + b_ref[...]


def _gln(p, x):
    n, d = x.shape
    return pl.pallas_call(
        _gln_body,
        in_specs=[
            pl.BlockSpec((n, d), lambda: (0, 0)),
            pl.BlockSpec((1, d), lambda: (0, 0)),
            pl.BlockSpec((1, d), lambda: (0, 0)),
        ],
        out_specs=pl.BlockSpec((n, d), lambda: (0, 0)),
        out_shape=jax.ShapeDtypeStruct((n, d), jnp.float32),
    )(x, p["g"].reshape(1, d), p["b"].reshape(1, d))


def _split2_body(x_ref, o_ref):
    o_ref[0, :, :] = x_ref[:, 0:128]
    o_ref[1, :, :] = x_ref[:, 128:256]


def _split2(x):
    n = x.shape[0]
    return pl.pallas_call(
        _split2_body,
        grid=(n // ROW_BLOCK,),
        in_specs=[pl.BlockSpec((ROW_BLOCK, 256), lambda i: (i, 0))],
        out_specs=pl.BlockSpec((2, ROW_BLOCK, 128), lambda i: (0, i, 0)),
        out_shape=jax.ShapeDtypeStruct((2, n, 128), jnp.float32),
    )(x)


def _sage_lin_body(s_ref, c_ref, x_ref, wl_ref, wr_ref, b_ref, o_ref):
    cnt = c_ref[0, :, 0:1] + c_ref[1, :, 0:1]
    inv = 1.0 / jnp.maximum(cnt, 1.0)
    agg = jnp.concatenate([s_ref[0], s_ref[1]], axis=1) * inv
    o_ref[...] = (
        jnp.dot(agg, wl_ref[...], preferred_element_type=jnp.float32)
        + jnp.dot(x_ref[...], wr_ref[...], preferred_element_type=jnp.float32)
        + b_ref[...]
    )


def _sage_linear(s2, cnt2, x, wl, wr, b):
    n, k = x.shape
    p = wl.shape[1]
    grid = (n // ROW_BLOCK,)
    return pl.pallas_call(
        _sage_lin_body,
        grid=grid,
        in_specs=[
            pl.BlockSpec((2, ROW_BLOCK, 128), lambda i: (0, i, 0)),
            pl.BlockSpec((2, ROW_BLOCK, 128), lambda i: (0, i, 0)),
            pl.BlockSpec((ROW_BLOCK, k), lambda i: (i, 0)),
            pl.BlockSpec((k, p), lambda i: (0, 0)),
            pl.BlockSpec((k, p), lambda i: (0, 0)),
            pl.BlockSpec((1, p), lambda i: (0, 0)),
        ],
        out_specs=pl.BlockSpec((ROW_BLOCK, p), lambda i: (i, 0)),
        out_shape=jax.ShapeDtypeStruct((n, p), jnp.float32),
    )(s2, cnt2, x, wl, wr, b.reshape(1, p))


def _sage(p, x_src, x_dst, ei, num_dst, cnt2, x2=None):
    if x2 is None:
        x2 = _split2(x_src)
    s2 = _sage_agg(x2, ei[0], ei[1])
    return _sage_linear(s2, cnt2, x_dst, p["l"]["W"], p["Wr"], p["l"]["b"])


def _proj_body(x_ref, w_ref, b_ref, o_ref, *, scale):
    o_ref[0, :, :] = (
        jnp.dot(x_ref[...], w_ref[...], preferred_element_type=jnp.float32)
        + b_ref[...]
    ) * scale


def _proj8(x, w, b, scale=1.0):
    n = x.shape[0]
    return pl.pallas_call(
        functools.partial(_proj_body, scale=scale),
        grid=(n // ROW_BLOCK, 8),
        in_specs=[
            pl.BlockSpec((ROW_BLOCK, 256), lambda i, j: (i, 0)),
            pl.BlockSpec((256, 128), lambda i, j: (0, j)),
            pl.BlockSpec((1, 128), lambda i, j: (0, j)),
        ],
        out_specs=pl.BlockSpec((1, ROW_BLOCK, 128), lambda i, j: (j, i, 0)),
        out_shape=jax.ShapeDtypeStruct((8, n, 128), jnp.float32),
    )(x, w, b.reshape(1, 1024))


def _mred_body(t_ref, o_ref):
    mx = jnp.max(t_ref[...], axis=0, keepdims=True)
    o_ref[...] = jnp.broadcast_to(mx, (8, TBL))


def _mred(tables):
    return pl.pallas_call(
        _mred_body,
        in_specs=[pl.BlockSpec((32, TBL), lambda: (0, 0))],
        out_specs=pl.BlockSpec((8, TBL), lambda: (0, 0)),
        out_shape=jax.ShapeDtypeStruct((8, TBL), jnp.float32),
    )(tables)


def _dadd_body(d_ref, o_ref):
    o_ref[...] = d_ref[0] + d_ref[1]


def _dadd(den2):
    return pl.pallas_call(
        _dadd_body,
        grid=(N_NODES // ROW_BLOCK,),
        in_specs=[pl.BlockSpec((2, ROW_BLOCK, 128), lambda i: (0, i, 0))],
        out_specs=pl.BlockSpec((ROW_BLOCK, 128), lambda i: (i, 0)),
        out_shape=jax.ShapeDtypeStruct((N_NODES, 128), jnp.float32),
    )(den2)


def _cat_body(o8_ref, wc_ref, x_ref, ws_ref, bc_ref, o_ref):
    acc = bc_ref[...]
    acc = acc + jnp.dot(x_ref[...], ws_ref[...],
                        preferred_element_type=jnp.float32)
    for b in range(8):
        acc = acc + jnp.dot(o8_ref[b], wc_ref[b],
                            preferred_element_type=jnp.float32)
    o_ref[...] = acc


def _cat_linear(o8, wcat_r, x, wsc, bc):
    n = x.shape[0]
    return pl.pallas_call(
        _cat_body,
        grid=(n // ROW_BLOCK,),
        in_specs=[
            pl.BlockSpec((8, ROW_BLOCK, 128), lambda i: (0, i, 0)),
            pl.BlockSpec((8, 128, 256), lambda i: (0, 0, 0)),
            pl.BlockSpec((ROW_BLOCK, 256), lambda i: (i, 0)),
            pl.BlockSpec((256, 256), lambda i: (0, 0)),
            pl.BlockSpec((1, 256), lambda i: (0, 0)),
        ],
        out_specs=pl.BlockSpec((ROW_BLOCK, 256), lambda i: (i, 0)),
        out_shape=jax.ShapeDtypeStruct((n, 256), jnp.float32),
    )(o8, wcat_r, x, wsc, bc)


def _mm_small_body(a_ref, b_ref, o_ref):
    o_ref[...] = jnp.dot(a_ref[...], b_ref[...],
                         preferred_element_type=jnp.float32)


def _mm_small(a, b):
    m, k = a.shape
    p = b.shape[1]
    return pl.pallas_call(
        _mm_small_body,
        in_specs=[
            pl.BlockSpec((m, k), lambda: (0, 0)),
            pl.BlockSpec((k, p), lambda: (0, 0)),
        ],
        out_specs=pl.BlockSpec((m, p), lambda: (0, 0)),
        out_shape=jax.ShapeDtypeStruct((m, p), jnp.float32),
    )(a, b)


def _tconv_cat(pa, pcat, x_src, x_dst, ei):
    src, dst = ei[0], ei[1]
    q8 = _proj8(x_dst, pa["q"]["W"], pa["q"]["b"], scale=1.0 / 16.0)
    k8 = _proj8(x_src, pa["k"]["W"], pa["k"]["b"])
    v8 = _proj8(x_src, pa["v"]["W"], pa["v"]["b"])
    lg, tables = _attn_logits(q8, k8, src, dst)
    m = _mred(tables)
    ex, den2 = _attn_den(lg, m, src, dst)
    den = _dadd(den2)
    o8 = _attn_vagg(v8, ex, den, src, dst)
    skw = jnp.concatenate(
        [pa["skip"]["W"], pa["skip"]["b"].reshape(1, -1),
         jnp.zeros((7, HEADS * HID), jnp.float32)], axis=0)
    fold = _mm_small(skw, pcat["W"])
    wsc = fold[:256]
    bc = (fold[256] + pcat["b"]).reshape(1, 256)
    wcat_r = pcat["W"].reshape(8, 128, 256)
    return _cat_linear(o8, wcat_r, x_dst, wsc, bc)


def _attention(p, x, ei, n):
    return _gln(p["norm"], _tconv_cat(p["att"], p["cat"], x, x, ei))


def _ffw(p, x, ei, n, cnt2):
    return _gln(p["norm"], _sage(p["sage"], x, x, ei, n, cnt2))


def kernel(ast_x, llc_x, params, ast_edge_index, llc_edge_index):
    n = N_NODES
    ast_edge_index = jnp.pad(ast_edge_index, ((0, 0), (0, 16)))
    llc_edge_index = jnp.pad(llc_edge_index, ((0, 0), (0, 16)))
    cnt_llc = _cnt_kernel(llc_edge_index[1])
    cnt_ast = _cnt_kernel(ast_edge_index[1])

    x = _sage(params["enc"]["embed"], llc_x, llc_x, llc_edge_index, n, cnt_llc)
    for u in params["enc"]["units"]:
        x = _attention(u["att"], x, llc_edge_index, n)
        x = _ffw(u["ffw"], x, llc_edge_index, n, cnt_llc)
    enc_out = x

    y = _sage(params["dec"]["embed"], ast_x, ast_x, ast_edge_index, n, cnt_ast)
    for u in params["dec"]["units"]:
        y = _attention(u["ast_att"], y, ast_edge_index, n)
        y = _gln(u["norm"],
                 _tconv_cat(u["cross"], u["cat"], y, enc_out, ast_edge_index))
        y = _ffw(u["ffw"], y, ast_edge_index, n, cnt_ast)

    nn_w, nn_b = _pad_w(params["new_node"]["W"], params["new_node"]["b"])
    new_node = _matmul_log_softmax(y, nn_w, nn_b, params["new_node"]["W"].shape[1])

    ns = params["node_sel"]
    s2 = _sage_agg(_split2(y), ast_edge_index[0], ast_edge_index[1])
    wl, bp = _pad_w(ns["l"]["W"], ns["l"]["b"])
    wr, _ = _pad_w(ns["Wr"], ns["l"]["b"])
    node_sel = _sage_linear(s2, cnt_ast, y, wl, wr, bp)[:, :1]
    return new_node, node_sel
